# baseline (speedup 1.0000x reference)
import functools

import jax
import jax.numpy as jnp
from jax import lax
from jax.experimental import pallas as pl
from jax.experimental.pallas import tpu as pltpu
from jax.experimental.pallas import tpu_sc as plsc

D_MODEL = 64
CB = 256
BW = 512
N_CHUNKS = 100


@functools.lru_cache(maxsize=None)
def _make_lookup(batch: int, hist: int, vocab: int, d: int):
    info = plsc.get_sparse_core_info()
    nc, ns = info.num_cores, info.num_subcores
    nw = nc * ns
    assert batch == nw * BW and d == D_MODEL and BW == 2 * CB
    n_chunks = hist * 2

    mesh = plsc.VectorSubcoreMesh(core_axis_name="c", subcore_axis_name="s")

    @functools.partial(
        pl.kernel,
        mesh=mesh,
        out_type=jax.ShapeDtypeStruct((hist, d, batch), jnp.float32),
        compiler_params=pltpu.CompilerParams(
            use_tc_tiling_on_sc=True, needs_layout_passes=False),
        scratch_types=[
            [pltpu.VMEM((CB,), jnp.int32)] * 2,
            pltpu.VMEM((2, CB), jnp.int32),
            pltpu.VMEM((2, CB, 128), jnp.float32),
            pltpu.VMEM((2, d, CB), jnp.float32),
            [pltpu.SemaphoreType.DMA] * 2,
            [pltpu.SemaphoreType.DMA] * 2,
            [pltpu.SemaphoreType.DMA] * 2,
            [pltpu.SemaphoreType.DMA] * 2,
        ],
    )
    def lookup_kernel(xsh_hbm, h64_hbm, tab_hbm, out_hbm,
                      idx_v, h64_v, g_v, t_v, isems, hsems, gsems, ssems):
        wid = lax.axis_index("s") * nc + lax.axis_index("c")
        wbase = wid * BW

        def chunk_hb(k):
            return k // 2, wbase + (k % 2) * CB

        def i_copy(k, b):
            h, b0 = chunk_hb(k)
            return pltpu.make_async_copy(
                xsh_hbm.at[h, pl.ds(b0, CB)], idx_v[b], isems[b])

        def h_copy(k, b):
            h, b0 = chunk_hb(k)
            return pltpu.make_async_copy(
                h64_hbm.at[h, pl.ds(b0, CB)], h64_v.at[b], hsems[b])

        def g_copy(k, b):
            return pltpu.make_async_copy(
                tab_hbm.at[idx_v[b]], g_v.at[b], gsems[b])

        def s_copy(k, b):
            h, b0 = chunk_hb(k)
            return pltpu.make_async_copy(
                t_v.at[b], out_hbm.at[h, :, pl.ds(b0, CB)], ssems[b])

        def transpose(b):
            def cb_body(cb, carry):
                rows = lax.iota(jnp.int32, 16) + cb * 16
                hv = h64_v[b, pl.ds(cb * 16, 16)]
                for j in range(d):
                    col = hv + j
                    v = plsc.load_gather(g_v.at[b], [rows, col])
                    t_v[b, j, pl.ds(cb * 16, 16)] = v
                return carry
            lax.fori_loop(0, CB // 16, cb_body, 0)

        i_copy(0, 0).start()
        h_copy(0, 0).start()
        i_copy(0, 0).wait()
        g_copy(0, 0).start()
        i_copy(1, 1).start()
        h_copy(1, 1).start()

        def body(kk, carry):
            for b in range(2):
                k = kk * 2 + b
                nb = 1 - b

                @pl.when(k + 1 < n_chunks)
                def _():
                    i_copy(k + 1, nb).wait()
                    g_copy(k + 1, nb).start()

                g_copy(k, b).wait()
                h_copy(k, b).wait()

                @pl.when(k >= 2)
                def _():
                    s_copy(k - 2, b).wait()

                transpose(b)
                s_copy(k, b).start()

                @pl.when(k + 2 < n_chunks)
                def _():
                    i_copy(k + 2, b).start()
                    h_copy(k + 2, b).start()

            return carry

        lax.fori_loop(0, n_chunks // 2, body, 0)

        s_copy(n_chunks - 2, 0).wait()
        s_copy(n_chunks - 1, 1).wait()

    return lookup_kernel


def kernel(x, table):
    batch, hist = x.shape
    vocab, d = table.shape
    xi = x.astype(jnp.int32)
    xsh = (xi >> 1).T
    h64 = ((xi & 1) << 6).T
    tab = table.reshape(vocab // 2, 2 * d)
    out = _make_lookup(batch, hist, vocab, d)(xsh, h64, tab)
    return out.transpose(2, 0, 1)


# speedup vs baseline: 1.4037x; 1.4037x over previous
import functools

import jax
import jax.numpy as jnp
from jax import lax
from jax.experimental import pallas as pl
from jax.experimental.pallas import tpu as pltpu
from jax.experimental.pallas import tpu_sc as plsc

D_MODEL = 64
CB = 256
BW = 512
N_CHUNKS = 100


@functools.lru_cache(maxsize=None)
def _make_lookup(batch: int, hist: int, vocab: int, d: int):
    info = plsc.get_sparse_core_info()
    nc, ns = info.num_cores, info.num_subcores
    nw = nc * ns
    assert batch == nw * BW and d == D_MODEL and BW == 2 * CB
    n_chunks = hist * 2

    mesh = plsc.VectorSubcoreMesh(core_axis_name="c", subcore_axis_name="s")

    @functools.partial(
        pl.kernel,
        mesh=mesh,
        out_type=jax.ShapeDtypeStruct((hist, d, batch), jnp.float32),
        compiler_params=pltpu.CompilerParams(
            use_tc_tiling_on_sc=True, needs_layout_passes=False),
        scratch_types=[
            [pltpu.VMEM((CB,), jnp.int32)] * 2,
            pltpu.VMEM((2, CB), jnp.int32),
            pltpu.VMEM((2, CB, 128), jnp.float32),
            pltpu.VMEM((2, d, CB), jnp.float32),
            [pltpu.SemaphoreType.DMA] * 2,
            [pltpu.SemaphoreType.DMA] * 2,
            [pltpu.SemaphoreType.DMA] * 2,
            [pltpu.SemaphoreType.DMA] * 2,
        ],
    )
    def lookup_kernel(xsh_hbm, h64_hbm, tab_hbm, out_hbm,
                      idx_v, h64_v, g_v, t_v, isems, hsems, gsems, ssems):
        wid = lax.axis_index("s") * nc + lax.axis_index("c")
        wbase = wid * BW

        def chunk_hb(k):
            return k // 2, wbase + (k % 2) * CB

        def i_copy(k, b):
            h, b0 = chunk_hb(k)
            return pltpu.make_async_copy(
                xsh_hbm.at[h, pl.ds(b0, CB)], idx_v[b], isems[b])

        def h_copy(k, b):
            h, b0 = chunk_hb(k)
            return pltpu.make_async_copy(
                h64_hbm.at[h, pl.ds(b0, CB)], h64_v.at[b], hsems[b])

        def g_copy(k, b):
            return pltpu.make_async_copy(
                tab_hbm.at[idx_v[b]], g_v.at[b], gsems[b])

        def s_copy(k, b):
            h, b0 = chunk_hb(k)
            return pltpu.make_async_copy(
                t_v.at[b], out_hbm.at[h, :, pl.ds(b0, CB)], ssems[b])

        def transpose(b):
            depth = 4

            def cb_body(cb, carry):
                rows = lax.iota(jnp.int32, 16) + cb * 16
                hv = h64_v[b, pl.ds(cb * 16, 16)]
                pend = [plsc.load_gather(g_v.at[b], [rows, hv + j])
                        for j in range(depth)]
                for j in range(depth, d):
                    v = plsc.load_gather(g_v.at[b], [rows, hv + j])
                    t_v[b, j - depth, pl.ds(cb * 16, 16)] = pend[0]
                    pend = pend[1:] + [v]
                for j in range(depth):
                    t_v[b, d - depth + j, pl.ds(cb * 16, 16)] = pend[j]
                return carry

            lax.fori_loop(0, CB // 16, cb_body, 0)

        i_copy(0, 0).start()
        h_copy(0, 0).start()
        i_copy(0, 0).wait()
        g_copy(0, 0).start()
        i_copy(1, 1).start()
        h_copy(1, 1).start()

        def body(kk, carry):
            for b in range(2):
                k = kk * 2 + b
                nb = 1 - b

                @pl.when(k + 1 < n_chunks)
                def _():
                    i_copy(k + 1, nb).wait()
                    g_copy(k + 1, nb).start()

                g_copy(k, b).wait()
                h_copy(k, b).wait()

                @pl.when(k >= 2)
                def _():
                    s_copy(k - 2, b).wait()

                transpose(b)
                s_copy(k, b).start()

                @pl.when(k + 2 < n_chunks)
                def _():
                    i_copy(k + 2, b).start()
                    h_copy(k + 2, b).start()

            return carry

        lax.fori_loop(0, n_chunks // 2, body, 0)

        s_copy(n_chunks - 2, 0).wait()
        s_copy(n_chunks - 1, 1).wait()

    return lookup_kernel


def kernel(x, table):
    batch, hist = x.shape
    vocab, d = table.shape
    xi = x.astype(jnp.int32)
    xsh = (xi >> 1).T
    h64 = ((xi & 1) << 6).T
    tab = table.reshape(vocab // 2, 2 * d)
    out = _make_lookup(batch, hist, vocab, d)(xsh, h64, tab)
    return out.transpose(2, 0, 1)


# speedup vs baseline: 1.5709x; 1.1191x over previous
import functools

import jax
import jax.numpy as jnp
from jax import lax
from jax.experimental import pallas as pl
from jax.experimental.pallas import tpu as pltpu
from jax.experimental.pallas import tpu_sc as plsc

D_MODEL = 64
CB = 256
BW = 512
N_CHUNKS = 100


@functools.lru_cache(maxsize=None)
def _make_lookup(batch: int, hist: int, vocab: int, d: int):
    info = plsc.get_sparse_core_info()
    nc, ns = info.num_cores, info.num_subcores
    nw = nc * ns
    assert batch == nw * BW and d == D_MODEL and BW == 2 * CB
    n_chunks = hist * 2

    mesh = plsc.VectorSubcoreMesh(core_axis_name="c", subcore_axis_name="s")

    @functools.partial(
        pl.kernel,
        mesh=mesh,
        out_type=jax.ShapeDtypeStruct((hist, d, batch), jnp.float32),
        compiler_params=pltpu.CompilerParams(
            use_tc_tiling_on_sc=True, needs_layout_passes=False),
        scratch_types=[
            [pltpu.VMEM((CB,), jnp.int32)] * 2,
            pltpu.VMEM((2, CB), jnp.int32),
            pltpu.VMEM((2, CB, 128), jnp.float32),
            pltpu.VMEM((2, d, CB), jnp.float32),
            [pltpu.SemaphoreType.DMA] * 2,
            [pltpu.SemaphoreType.DMA] * 2,
            [pltpu.SemaphoreType.DMA] * 2,
            [pltpu.SemaphoreType.DMA] * 2,
        ],
    )
    def lookup_kernel(xsh_hbm, h64_hbm, tab_hbm, out_hbm,
                      idx_v, h64_v, g_v, t_v, isems, hsems, gsems, ssems):
        wid = lax.axis_index("s") * nc + lax.axis_index("c")
        wbase = wid * BW

        def chunk_hb(k):
            return k // 2, wbase + (k % 2) * CB

        def i_copy(k, b):
            h, b0 = chunk_hb(k)
            return pltpu.make_async_copy(
                xsh_hbm.at[h, pl.ds(b0, CB)], idx_v[b], isems[b])

        def h_copy(k, b):
            h, b0 = chunk_hb(k)
            return pltpu.make_async_copy(
                h64_hbm.at[h, pl.ds(b0, CB)], h64_v.at[b], hsems[b])

        def g_copy(k, b):
            return pltpu.make_async_copy(
                tab_hbm.at[idx_v[b]], g_v.at[b], gsems[b])

        def s_copy(k, b):
            h, b0 = chunk_hb(k)
            return pltpu.make_async_copy(
                t_v.at[b], out_hbm.at[h, :, pl.ds(b0, CB)], ssems[b])

        def transpose(b):
            depth = 4

            def cb_body(cb, carry):
                rows = lax.iota(jnp.int32, 16) + cb * 16
                hv = h64_v[b, pl.ds(cb * 16, 16)]
                pend = [plsc.load_gather(g_v.at[b], [rows, hv + j])
                        for j in range(depth)]
                for j in range(depth, d):
                    v = plsc.load_gather(g_v.at[b], [rows, hv + j])
                    t_v[b, j - depth, pl.ds(cb * 16, 16)] = pend[0]
                    pend = pend[1:] + [v]
                for j in range(depth):
                    t_v[b, d - depth + j, pl.ds(cb * 16, 16)] = pend[j]
                return carry

            lax.fori_loop(0, CB // 16, cb_body, 0)

        i_copy(0, 0).start()
        h_copy(0, 0).start()
        i_copy(0, 0).wait()
        g_copy(0, 0).start()
        i_copy(1, 1).start()
        h_copy(1, 1).start()

        def body(kk, carry):
            for b in range(2):
                k = kk * 2 + b
                nb = 1 - b

                @pl.when(k + 1 < n_chunks)
                def _():
                    i_copy(k + 1, nb).wait()
                    g_copy(k + 1, nb).start()

                g_copy(k, b).wait()
                h_copy(k, b).wait()

                @pl.when(k >= 2)
                def _():
                    s_copy(k - 2, b).wait()

                transpose(b)
                s_copy(k, b).start()

                @pl.when(k + 2 < n_chunks)
                def _():
                    i_copy(k + 2, b).start()
                    h_copy(k + 2, b).start()

            return carry

        lax.fori_loop(0, n_chunks // 2, body, 0)

        s_copy(n_chunks - 2, 0).wait()
        s_copy(n_chunks - 1, 1).wait()

    return lookup_kernel


_TBLK = 1024


@functools.lru_cache(maxsize=None)
def _make_pairer(vocab: int, d: int):
    n_blk = -(-vocab // (2 * _TBLK))

    def body(in_ref, out_ref):
        a = in_ref[...]
        out_ref[...] = jnp.concatenate(
            [a[:, :_TBLK].T, a[:, _TBLK:].T], axis=1)

    return pl.pallas_call(
        body,
        grid=(n_blk,),
        in_specs=[pl.BlockSpec((d, 2 * _TBLK), lambda i: (0, i))],
        out_specs=pl.BlockSpec((_TBLK, 2 * d), lambda i: (i, 0)),
        out_shape=jax.ShapeDtypeStruct((n_blk * _TBLK, 2 * d), jnp.float32),
    )


def kernel(x, table):
    batch, hist = x.shape
    vocab, d = table.shape
    xi = x.astype(jnp.int32)
    xsh = ((xi // (2 * _TBLK)) * _TBLK + xi % _TBLK).T
    h64 = (((xi // _TBLK) & 1) << 6).T
    tab = _make_pairer(vocab, d)(table.T)
    out = _make_lookup(batch, hist, vocab, d)(xsh, h64, tab)
    return out.transpose(2, 0, 1)


# speedup vs baseline: 2.2246x; 1.4161x over previous
import functools

import jax
import jax.numpy as jnp
from jax import lax
from jax.experimental import pallas as pl
from jax.experimental.pallas import tpu as pltpu
from jax.experimental.pallas import tpu_sc as plsc

D_MODEL = 64
CB = 256
BW = 512
N_CHUNKS = 100


@functools.lru_cache(maxsize=None)
def _make_lookup(batch: int, hist: int, vocab: int, d: int):
    info = plsc.get_sparse_core_info()
    nc, ns = info.num_cores, info.num_subcores
    nw = nc * ns
    assert batch == nw * BW and d == D_MODEL and BW == 2 * CB
    n_chunks = hist * 2

    mesh = plsc.VectorSubcoreMesh(core_axis_name="c", subcore_axis_name="s")

    @functools.partial(
        pl.kernel,
        mesh=mesh,
        out_type=jax.ShapeDtypeStruct((hist, d, batch), jnp.float32),
        compiler_params=pltpu.CompilerParams(
            use_tc_tiling_on_sc=True, needs_layout_passes=False),
        scratch_types=[
            [pltpu.VMEM((CB,), jnp.int32)] * 2,
            pltpu.VMEM((2, CB), jnp.int32),
            pltpu.VMEM((2, CB, 128), jnp.float32),
            pltpu.VMEM((2, d, CB + 2), jnp.float32),
            [pltpu.SemaphoreType.DMA] * 2,
            [pltpu.SemaphoreType.DMA] * 2,
            [pltpu.SemaphoreType.DMA] * 2,
            [pltpu.SemaphoreType.DMA] * 2,
        ],
    )
    def lookup_kernel(xsh_hbm, h64_hbm, tab_hbm, out_hbm,
                      idx_v, h64_v, g_v, t_v, isems, hsems, gsems, ssems):
        wid = lax.axis_index("s") * nc + lax.axis_index("c")
        wbase = wid * BW

        def chunk_hb(k):
            return k // 2, wbase + (k % 2) * CB

        def i_copy(k, b):
            h, b0 = chunk_hb(k)
            return pltpu.make_async_copy(
                xsh_hbm.at[h, pl.ds(b0, CB)], idx_v[b], isems[b])

        def h_copy(k, b):
            h, b0 = chunk_hb(k)
            return pltpu.make_async_copy(
                h64_hbm.at[h, pl.ds(b0, CB)], h64_v.at[b], hsems[b])

        def g_copy(k, b):
            return pltpu.make_async_copy(
                tab_hbm.at[idx_v[b]], g_v.at[b], gsems[b])

        def s_copy(k, b):
            h, b0 = chunk_hb(k)
            return pltpu.make_async_copy(
                t_v.at[b, :, pl.ds(0, CB)],
                out_hbm.at[h, :, pl.ds(b0, CB)], ssems[b])

        def transpose(b):
            depth = 4
            lane = lax.iota(jnp.int32, 16)

            def cb_body(cb, carry):
                colv = lane + cb * 16
                hv = h64_v[b, pl.ds(cb * 16, 16)]

                def loadj(j):
                    rowv = (lane + j) & (d - 1)
                    return rowv, plsc.load_gather(g_v.at[b], [colv, hv + rowv])

                def storej(rv):
                    rowv, v = rv
                    plsc.store_scatter(t_v.at[b], [rowv, colv], v)

                pend = [loadj(j) for j in range(depth)]
                for j in range(depth, d):
                    nxt = loadj(j)
                    storej(pend[0])
                    pend = pend[1:] + [nxt]
                for rv in pend:
                    storej(rv)
                return carry

            lax.fori_loop(0, CB // 16, cb_body, 0)

        i_copy(0, 0).start()
        h_copy(0, 0).start()
        i_copy(0, 0).wait()
        g_copy(0, 0).start()
        i_copy(1, 1).start()
        h_copy(1, 1).start()

        def body(kk, carry):
            for b in range(2):
                k = kk * 2 + b
                nb = 1 - b

                @pl.when(k + 1 < n_chunks)
                def _():
                    i_copy(k + 1, nb).wait()
                    g_copy(k + 1, nb).start()

                g_copy(k, b).wait()
                h_copy(k, b).wait()

                @pl.when(k >= 2)
                def _():
                    s_copy(k - 2, b).wait()

                transpose(b)
                s_copy(k, b).start()

                @pl.when(k + 2 < n_chunks)
                def _():
                    i_copy(k + 2, b).start()
                    h_copy(k + 2, b).start()

            return carry

        lax.fori_loop(0, n_chunks // 2, body, 0)

        s_copy(n_chunks - 2, 0).wait()
        s_copy(n_chunks - 1, 1).wait()

    return lookup_kernel


_TBLK = 1024


@functools.lru_cache(maxsize=None)
def _make_pairer(vocab: int, d: int):
    n_blk = -(-vocab // (2 * _TBLK))

    def body(in_ref, out_ref):
        a = in_ref[...]
        out_ref[...] = jnp.concatenate(
            [a[:, :_TBLK].T, a[:, _TBLK:].T], axis=1)

    return pl.pallas_call(
        body,
        grid=(n_blk,),
        in_specs=[pl.BlockSpec((d, 2 * _TBLK), lambda i: (0, i))],
        out_specs=pl.BlockSpec((_TBLK, 2 * d), lambda i: (i, 0)),
        out_shape=jax.ShapeDtypeStruct((n_blk * _TBLK, 2 * d), jnp.float32),
    )


def kernel(x, table):
    batch, hist = x.shape
    vocab, d = table.shape
    xi = x.astype(jnp.int32)
    xsh = ((xi // (2 * _TBLK)) * _TBLK + xi % _TBLK).T
    h64 = (((xi // _TBLK) & 1) << 6).T
    tab = _make_pairer(vocab, d)(table.T)
    out = _make_lookup(batch, hist, vocab, d)(xsh, h64, tab)
    return out.transpose(2, 0, 1)


# speedup vs baseline: 2.8567x; 1.2841x over previous
import functools

import jax
import jax.numpy as jnp
from jax import lax
from jax.experimental import pallas as pl
from jax.experimental.pallas import tpu as pltpu
from jax.experimental.pallas import tpu_sc as plsc

D_MODEL = 64
CB = 256
BW = 512
N_CHUNKS = 100


@functools.lru_cache(maxsize=None)
def _make_lookup(batch: int, hist: int, vocab: int, d: int):
    info = plsc.get_sparse_core_info()
    nc, ns = info.num_cores, info.num_subcores
    nw = nc * ns
    assert batch == nw * BW and d == D_MODEL and BW == 2 * CB
    n_chunks = hist * 2

    mesh = plsc.VectorSubcoreMesh(core_axis_name="c", subcore_axis_name="s")

    @functools.partial(
        pl.kernel,
        mesh=mesh,
        out_type=jax.ShapeDtypeStruct((hist, d, batch), jnp.float32),
        compiler_params=pltpu.CompilerParams(
            use_tc_tiling_on_sc=True, needs_layout_passes=False),
        scratch_types=[
            [pltpu.VMEM((CB,), jnp.int32)] * 2,
            pltpu.VMEM((2, CB), jnp.int32),
            pltpu.VMEM((2, CB, 128), jnp.float32),
            pltpu.VMEM((2, d, CB + 2), jnp.float32),
            [pltpu.SemaphoreType.DMA] * 2,
            [pltpu.SemaphoreType.DMA] * 2,
            [pltpu.SemaphoreType.DMA] * 2,
            [pltpu.SemaphoreType.DMA] * 2,
        ],
    )
    def lookup_kernel(xsh_hbm, h64_hbm, tab_hbm, out_hbm,
                      idx_v, h64_v, g_v, t_v, isems, hsems, gsems, ssems):
        wid = lax.axis_index("s") * nc + lax.axis_index("c")
        wbase = wid * BW

        def chunk_hb(k):
            return k // 2, wbase + (k % 2) * CB

        def i_copy(k, b):
            h, b0 = chunk_hb(k)
            return pltpu.make_async_copy(
                xsh_hbm.at[h, pl.ds(b0, CB)], idx_v[b], isems[b])

        def h_copy(k, b):
            h, b0 = chunk_hb(k)
            return pltpu.make_async_copy(
                h64_hbm.at[h, pl.ds(b0, CB)], h64_v.at[b], hsems[b])

        def g_copy(k, b):
            return pltpu.make_async_copy(
                tab_hbm.at[idx_v[b]], g_v.at[b], gsems[b])

        def s_copy(k, b):
            h, b0 = chunk_hb(k)
            return pltpu.make_async_copy(
                t_v.at[b, :, pl.ds(0, CB)],
                out_hbm.at[h, :, pl.ds(b0, CB)], ssems[b])

        def transpose(b):
            depth = 4
            lane = lax.iota(jnp.int32, 16)

            def cb_body(cb, carry):
                colv = lane + cb * 16
                hv = h64_v[b, pl.ds(cb * 16, 16)]

                def loadj(j):
                    rowv = (lane + j) & (d - 1)
                    return rowv, plsc.load_gather(g_v.at[b], [colv, hv + rowv])

                def storej(rv):
                    rowv, v = rv
                    plsc.store_scatter(t_v.at[b], [rowv, colv], v)

                pend = [loadj(j) for j in range(depth)]
                for j in range(depth, d):
                    nxt = loadj(j)
                    storej(pend[0])
                    pend = pend[1:] + [nxt]
                for rv in pend:
                    storej(rv)
                return carry

            lax.fori_loop(0, CB // 16, cb_body, 0)

        i_copy(0, 0).start()
        h_copy(0, 0).start()
        i_copy(0, 0).wait()
        g_copy(0, 0).start()
        i_copy(1, 1).start()
        h_copy(1, 1).start()

        def body(kk, carry):
            for b in range(2):
                k = kk * 2 + b
                nb = 1 - b

                @pl.when(k + 1 < n_chunks)
                def _():
                    i_copy(k + 1, nb).wait()
                    g_copy(k + 1, nb).start()

                g_copy(k, b).wait()
                h_copy(k, b).wait()

                @pl.when(k >= 2)
                def _():
                    s_copy(k - 2, b).wait()

                transpose(b)
                s_copy(k, b).start()

                @pl.when(k + 2 < n_chunks)
                def _():
                    i_copy(k + 2, b).start()
                    h_copy(k + 2, b).start()

            return carry

        lax.fori_loop(0, n_chunks // 2, body, 0)

        s_copy(n_chunks - 2, 0).wait()
        s_copy(n_chunks - 1, 1).wait()

    return lookup_kernel


_TBLK = 4096


@functools.lru_cache(maxsize=None)
def _make_pairer(vocab: int, d: int):
    n_blk = -(-vocab // (2 * _TBLK))

    def body(in_ref, out_ref):
        a = in_ref[...]
        out_ref[...] = jnp.concatenate(
            [a[:, :_TBLK].T, a[:, _TBLK:].T], axis=1)

    return pl.pallas_call(
        body,
        grid=(n_blk,),
        in_specs=[pl.BlockSpec((d, 2 * _TBLK), lambda i: (0, i))],
        out_specs=pl.BlockSpec((_TBLK, 2 * d), lambda i: (i, 0)),
        out_shape=jax.ShapeDtypeStruct((n_blk * _TBLK, 2 * d), jnp.float32),
    )


def kernel(x, table):
    batch, hist = x.shape
    vocab, d = table.shape
    xi = x.astype(jnp.int32)
    xsh = ((xi // (2 * _TBLK)) * _TBLK + xi % _TBLK).T
    h64 = (((xi // _TBLK) & 1) << 6).T
    tab = _make_pairer(vocab, d)(table.T)
    out = _make_lookup(batch, hist, vocab, d)(xsh, h64, tab)
    return out.transpose(2, 0, 1)


# speedup vs baseline: 2.9955x; 1.0486x over previous
import functools

import jax
import jax.numpy as jnp
from jax import lax
from jax.experimental import pallas as pl
from jax.experimental.pallas import tpu as pltpu
from jax.experimental.pallas import tpu_sc as plsc

D_MODEL = 64
CB = 256
BW = 512
N_CHUNKS = 100


@functools.lru_cache(maxsize=None)
def _make_lookup(batch: int, hist: int, vocab: int, d: int):
    info = plsc.get_sparse_core_info()
    nc, ns = info.num_cores, info.num_subcores
    nw = nc * ns
    assert batch == nw * BW and d == D_MODEL and BW == 2 * CB
    n_chunks = hist * 2

    mesh = plsc.VectorSubcoreMesh(core_axis_name="c", subcore_axis_name="s")

    @functools.partial(
        pl.kernel,
        mesh=mesh,
        out_type=jax.ShapeDtypeStruct((hist, d, batch), jnp.float32),
        compiler_params=pltpu.CompilerParams(
            use_tc_tiling_on_sc=True, needs_layout_passes=False),
        scratch_types=[
            [pltpu.VMEM((CB,), jnp.int32)] * 2,
            pltpu.VMEM((2, CB), jnp.int32),
            pltpu.VMEM((2, CB, 128), jnp.float32),
            pltpu.VMEM((2, d, CB + 2), jnp.float32),
            [pltpu.SemaphoreType.DMA] * 2,
            [pltpu.SemaphoreType.DMA] * 2,
            [pltpu.SemaphoreType.DMA] * 2,
            [pltpu.SemaphoreType.DMA] * 2,
        ],
    )
    def lookup_kernel(xsh_hbm, h64_hbm, tab_hbm, out_hbm,
                      idx_v, h64_v, g_v, t_v, isems, hsems, gsems, ssems):
        wid = lax.axis_index("s") * nc + lax.axis_index("c")
        wbase = wid * BW

        def chunk_hb(k):
            return k // 2, wbase + (k % 2) * CB

        def i_copy(k, b):
            h, b0 = chunk_hb(k)
            return pltpu.make_async_copy(
                xsh_hbm.at[h, pl.ds(b0, CB)], idx_v[b], isems[b])

        def h_copy(k, b):
            h, b0 = chunk_hb(k)
            return pltpu.make_async_copy(
                h64_hbm.at[h, pl.ds(b0, CB)], h64_v.at[b], hsems[b])

        def g_copy(k, b):
            return pltpu.make_async_copy(
                tab_hbm.at[idx_v[b]], g_v.at[b], gsems[b])

        def s_copy(k, b):
            h, b0 = chunk_hb(k)
            return pltpu.make_async_copy(
                t_v.at[b, :, pl.ds(0, CB)],
                out_hbm.at[h, :, pl.ds(b0, CB)], ssems[b])

        def transpose(b):
            depth = 4
            lane = lax.iota(jnp.int32, 16)

            def cb_body(cb, carry):
                colv = lane + cb * 16
                hv = h64_v[b, pl.ds(cb * 16, 16)]

                def loadj(j):
                    rowv = (lane + j) & (d - 1)
                    return rowv, plsc.load_gather(g_v.at[b], [colv, hv + rowv])

                def storej(rv):
                    rowv, v = rv
                    plsc.store_scatter(t_v.at[b], [rowv, colv], v)

                pend = [loadj(j) for j in range(depth)]
                for j in range(depth, d):
                    nxt = loadj(j)
                    storej(pend[0])
                    pend = pend[1:] + [nxt]
                for rv in pend:
                    storej(rv)
                return carry

            lax.fori_loop(0, CB // 16, cb_body, 0)

        i_copy(0, 0).start()
        h_copy(0, 0).start()
        i_copy(0, 0).wait()
        g_copy(0, 0).start()
        i_copy(1, 1).start()
        h_copy(1, 1).start()

        def body(kk, carry):
            for b in range(2):
                k = kk * 2 + b
                nb = 1 - b

                @pl.when(k + 1 < n_chunks)
                def _():
                    i_copy(k + 1, nb).wait()
                    g_copy(k + 1, nb).start()

                g_copy(k, b).wait()
                h_copy(k, b).wait()

                @pl.when(k >= 2)
                def _():
                    s_copy(k - 2, b).wait()

                transpose(b)
                s_copy(k, b).start()

                @pl.when(k + 2 < n_chunks)
                def _():
                    i_copy(k + 2, b).start()
                    h_copy(k + 2, b).start()

            return carry

        lax.fori_loop(0, n_chunks // 2, body, 0)

        s_copy(n_chunks - 2, 0).wait()
        s_copy(n_chunks - 1, 1).wait()

    return lookup_kernel


_TBLK = 8192


@functools.lru_cache(maxsize=None)
def _make_pairer(vocab: int, d: int):
    n_blk = -(-vocab // (2 * _TBLK))

    def body(in_ref, out_ref):
        a = in_ref[...]
        out_ref[...] = jnp.concatenate(
            [a[:, :_TBLK].T, a[:, _TBLK:].T], axis=1)

    return pl.pallas_call(
        body,
        grid=(n_blk,),
        in_specs=[pl.BlockSpec((d, 2 * _TBLK), lambda i: (0, i))],
        out_specs=pl.BlockSpec((_TBLK, 2 * d), lambda i: (i, 0)),
        out_shape=jax.ShapeDtypeStruct((n_blk * _TBLK, 2 * d), jnp.float32),
    )


def kernel(x, table):
    batch, hist = x.shape
    vocab, d = table.shape
    xi = x.astype(jnp.int32)
    xsh = ((xi // (2 * _TBLK)) * _TBLK + xi % _TBLK).T
    h64 = (((xi // _TBLK) & 1) << 6).T
    tab = _make_pairer(vocab, d)(table.T)
    out = _make_lookup(batch, hist, vocab, d)(xsh, h64, tab)
    return out.transpose(2, 0, 1)


# speedup vs baseline: 2.9975x; 1.0007x over previous
import functools

import jax
import jax.numpy as jnp
from jax import lax
from jax.experimental import pallas as pl
from jax.experimental.pallas import tpu as pltpu
from jax.experimental.pallas import tpu_sc as plsc

D_MODEL = 64
CB = 256
BW = 512
N_CHUNKS = 100


@functools.lru_cache(maxsize=None)
def _make_lookup(batch: int, hist: int, vocab: int, d: int):
    info = plsc.get_sparse_core_info()
    nc, ns = info.num_cores, info.num_subcores
    nw = nc * ns
    assert batch == nw * BW and d == D_MODEL and BW == 2 * CB
    n_chunks = hist * 2

    mesh = plsc.VectorSubcoreMesh(core_axis_name="c", subcore_axis_name="s")

    @functools.partial(
        pl.kernel,
        mesh=mesh,
        out_type=jax.ShapeDtypeStruct((hist, d, batch), jnp.float32),
        compiler_params=pltpu.CompilerParams(
            use_tc_tiling_on_sc=True, needs_layout_passes=False,
            disable_bounds_checks=True),
        scratch_types=[
            [pltpu.VMEM((CB,), jnp.int32)] * 2,
            pltpu.VMEM((2, CB), jnp.int32),
            pltpu.VMEM((2, CB, 128), jnp.float32),
            pltpu.VMEM((2, d, CB + 2), jnp.float32),
            [pltpu.SemaphoreType.DMA] * 2,
            [pltpu.SemaphoreType.DMA] * 2,
            [pltpu.SemaphoreType.DMA] * 2,
            [pltpu.SemaphoreType.DMA] * 2,
        ],
    )
    def lookup_kernel(xsh_hbm, h64_hbm, tab_hbm, out_hbm,
                      idx_v, h64_v, g_v, t_v, isems, hsems, gsems, ssems):
        wid = lax.axis_index("s") * nc + lax.axis_index("c")
        wbase = wid * BW

        def chunk_hb(k):
            return k // 2, wbase + (k % 2) * CB

        def i_copy(k, b):
            h, b0 = chunk_hb(k)
            return pltpu.make_async_copy(
                xsh_hbm.at[h, pl.ds(b0, CB)], idx_v[b], isems[b])

        def h_copy(k, b):
            h, b0 = chunk_hb(k)
            return pltpu.make_async_copy(
                h64_hbm.at[h, pl.ds(b0, CB)], h64_v.at[b], hsems[b])

        def g_copy(k, b):
            return pltpu.make_async_copy(
                tab_hbm.at[idx_v[b]], g_v.at[b], gsems[b])

        def s_copy(k, b):
            h, b0 = chunk_hb(k)
            return pltpu.make_async_copy(
                t_v.at[b, :, pl.ds(0, CB)],
                out_hbm.at[h, :, pl.ds(b0, CB)], ssems[b])

        def transpose(b):
            depth = 4
            lane = lax.iota(jnp.int32, 16)

            def cb_body(cb, carry):
                colv = lane + cb * 16
                hv = h64_v[b, pl.ds(cb * 16, 16)]

                def loadj(j):
                    rowv = (lane + j) & (d - 1)
                    return rowv, plsc.load_gather(g_v.at[b], [colv, hv + rowv])

                def storej(rv):
                    rowv, v = rv
                    plsc.store_scatter(t_v.at[b], [rowv, colv], v)

                pend = [loadj(j) for j in range(depth)]
                for j in range(depth, d):
                    nxt = loadj(j)
                    storej(pend[0])
                    pend = pend[1:] + [nxt]
                for rv in pend:
                    storej(rv)
                return carry

            lax.fori_loop(0, CB // 16, cb_body, 0)

        i_copy(0, 0).start()
        h_copy(0, 0).start()
        i_copy(0, 0).wait()
        g_copy(0, 0).start()
        i_copy(1, 1).start()
        h_copy(1, 1).start()

        def body(kk, carry):
            for b in range(2):
                k = kk * 2 + b
                nb = 1 - b

                @pl.when(k + 1 < n_chunks)
                def _():
                    i_copy(k + 1, nb).wait()
                    g_copy(k + 1, nb).start()

                g_copy(k, b).wait()
                h_copy(k, b).wait()

                @pl.when(k >= 2)
                def _():
                    s_copy(k - 2, b).wait()

                transpose(b)
                s_copy(k, b).start()

                @pl.when(k + 2 < n_chunks)
                def _():
                    i_copy(k + 2, b).start()
                    h_copy(k + 2, b).start()

            return carry

        lax.fori_loop(0, n_chunks // 2, body, 0)

        s_copy(n_chunks - 2, 0).wait()
        s_copy(n_chunks - 1, 1).wait()

    return lookup_kernel


_TBLK = 8192


@functools.lru_cache(maxsize=None)
def _make_pairer(vocab: int, d: int):
    n_blk = -(-vocab // (2 * _TBLK))

    def body(in_ref, out_ref):
        a = in_ref[...]
        eye = jax.lax.broadcasted_iota(jnp.int32, (d, d), 0)
        eye = (eye == jax.lax.broadcasted_iota(jnp.int32, (d, d), 1))
        eye = eye.astype(jnp.float32)
        t0 = jax.lax.dot_general(a[:, :_TBLK], eye, (((0,), (0,)), ((), ())))
        t1 = jax.lax.dot_general(a[:, _TBLK:], eye, (((0,), (0,)), ((), ())))
        out_ref[...] = jnp.concatenate([t0, t1], axis=1)

    return pl.pallas_call(
        body,
        grid=(n_blk,),
        in_specs=[pl.BlockSpec((d, 2 * _TBLK), lambda i: (0, i))],
        out_specs=pl.BlockSpec((_TBLK, 2 * d), lambda i: (i, 0)),
        out_shape=jax.ShapeDtypeStruct((n_blk * _TBLK, 2 * d), jnp.float32),
    )


def kernel(x, table):
    batch, hist = x.shape
    vocab, d = table.shape
    xi = x.astype(jnp.int32)
    xsh = ((xi // (2 * _TBLK)) * _TBLK + xi % _TBLK).T
    h64 = (((xi // _TBLK) & 1) << 6).T
    tab = _make_pairer(vocab, d)(table.T)
    out = _make_lookup(batch, hist, vocab, d)(xsh, h64, tab)
    return out.transpose(2, 0, 1)


# speedup vs baseline: 2.9977x; 1.0001x over previous
import functools

import jax
import jax.numpy as jnp
from jax import lax
from jax.experimental import pallas as pl
from jax.experimental.pallas import tpu as pltpu
from jax.experimental.pallas import tpu_sc as plsc

D_MODEL = 64
CB = 256
BW = 512
N_CHUNKS = 100


@functools.lru_cache(maxsize=None)
def _make_lookup(batch: int, hist: int, vocab: int, d: int):
    info = plsc.get_sparse_core_info()
    nc, ns = info.num_cores, info.num_subcores
    nw = nc * ns
    assert batch == nw * BW and d == D_MODEL and BW == 2 * CB
    n_chunks = hist * 2

    mesh = plsc.VectorSubcoreMesh(core_axis_name="c", subcore_axis_name="s")

    @functools.partial(
        pl.kernel,
        mesh=mesh,
        out_type=jax.ShapeDtypeStruct((hist, d, batch), jnp.float32),
        compiler_params=pltpu.CompilerParams(
            use_tc_tiling_on_sc=True, needs_layout_passes=False,
            disable_bounds_checks=True),
        scratch_types=[
            [pltpu.VMEM((CB,), jnp.int32)] * 2,
            pltpu.VMEM((2, CB), jnp.int32),
            pltpu.VMEM((2, CB, 128), jnp.float32),
            pltpu.VMEM((2, d, CB + 2), jnp.float32),
            [pltpu.SemaphoreType.DMA] * 2,
            [pltpu.SemaphoreType.DMA] * 2,
            [pltpu.SemaphoreType.DMA] * 2,
            [pltpu.SemaphoreType.DMA] * 2,
        ],
    )
    def lookup_kernel(xsh_hbm, h64_hbm, tab_hbm, out_hbm,
                      idx_v, h64_v, g_v, t_v, isems, hsems, gsems, ssems):
        wid = lax.axis_index("s") * nc + lax.axis_index("c")
        wbase = wid * BW

        def chunk_hb(k):
            return k // 2, wbase + (k % 2) * CB

        def i_copy(k, b):
            h, b0 = chunk_hb(k)
            return pltpu.make_async_copy(
                xsh_hbm.at[h, pl.ds(b0, CB)], idx_v[b], isems[b])

        def h_copy(k, b):
            h, b0 = chunk_hb(k)
            return pltpu.make_async_copy(
                h64_hbm.at[h, pl.ds(b0, CB)], h64_v.at[b], hsems[b])

        def g_copy(k, b):
            return pltpu.make_async_copy(
                tab_hbm.at[idx_v[b]], g_v.at[b], gsems[b])

        def s_copy(k, b):
            h, b0 = chunk_hb(k)
            return pltpu.make_async_copy(
                t_v.at[b, :, pl.ds(0, CB)],
                out_hbm.at[h, :, pl.ds(b0, CB)], ssems[b])

        def transpose(b):
            depth = 4
            lane = lax.iota(jnp.int32, 16)

            def cb_body(cb, carry):
                colv = lane + cb * 16
                hv = h64_v[b, pl.ds(cb * 16, 16)]

                def loadj(j):
                    rowv = (lane + j) & (d - 1)
                    return rowv, plsc.load_gather(g_v.at[b], [colv, hv + rowv])

                def storej(rv):
                    rowv, v = rv
                    plsc.store_scatter(t_v.at[b], [rowv, colv], v)

                pend = [loadj(j) for j in range(depth)]
                for j in range(depth, d):
                    nxt = loadj(j)
                    storej(pend[0])
                    pend = pend[1:] + [nxt]
                for rv in pend:
                    storej(rv)
                return carry

            lax.fori_loop(0, CB // 16, cb_body, 0)

        i_copy(0, 0).start()
        h_copy(0, 0).start()
        i_copy(0, 0).wait()
        g_copy(0, 0).start()
        i_copy(1, 1).start()
        h_copy(1, 1).start()

        def body(kk, carry):
            for b in range(2):
                k = kk * 2 + b
                nb = 1 - b

                @pl.when(k + 1 < n_chunks)
                def _():
                    i_copy(k + 1, nb).wait()
                    g_copy(k + 1, nb).start()

                g_copy(k, b).wait()
                h_copy(k, b).wait()

                @pl.when(k >= 2)
                def _():
                    s_copy(k - 2, b).wait()

                transpose(b)
                s_copy(k, b).start()

                @pl.when(k + 2 < n_chunks)
                def _():
                    i_copy(k + 2, b).start()
                    h_copy(k + 2, b).start()

            return carry

        lax.fori_loop(0, n_chunks // 2, body, 0)

        s_copy(n_chunks - 2, 0).wait()
        s_copy(n_chunks - 1, 1).wait()

    return lookup_kernel


_TBLK = 8192


@functools.lru_cache(maxsize=None)
def _make_pairer(vocab: int, d: int):
    n_blk = -(-vocab // (2 * _TBLK))

    def body(in_ref, out_ref):
        a = in_ref[...]
        out_ref[...] = jnp.concatenate(
            [a[:, :_TBLK].T, a[:, _TBLK:].T], axis=1)

    return pl.pallas_call(
        body,
        grid=(n_blk,),
        in_specs=[pl.BlockSpec((d, 2 * _TBLK), lambda i: (0, i))],
        out_specs=pl.BlockSpec((_TBLK, 2 * d), lambda i: (i, 0)),
        out_shape=jax.ShapeDtypeStruct((n_blk * _TBLK, 2 * d), jnp.float32),
    )


def kernel(x, table):
    batch, hist = x.shape
    vocab, d = table.shape
    xi = x.astype(jnp.int32)
    xsh = ((xi // (2 * _TBLK)) * _TBLK + xi % _TBLK).T
    h64 = (((xi // _TBLK) & 1) << 6).T
    tab = _make_pairer(vocab, d)(table.T)
    out = _make_lookup(batch, hist, vocab, d)(xsh, h64, tab)
    return out.transpose(2, 0, 1)
